# baseline (speedup 1.0000x reference)
import jax
import jax.numpy as jnp
from jax.experimental import pallas as pl
from jax.experimental.pallas import tpu as pltpu

N = 4096
D = 128
ALPHA = 40.0
BETA = 2.0
MARGIN = 0.5

R = 256
NJ = N // 128


def _wait_step(step, loss_hbm, grad_hbm, ls_ref, gs_ref, sem_l, sem_g):
    slot = jax.lax.rem(step, 2)
    row0 = step * R
    for j in range(NJ):
        pltpu.make_async_copy(
            ls_ref.at[slot, j], loss_hbm.at[pl.ds(row0, R), j], sem_l).wait()
        pltpu.make_async_copy(
            gs_ref.at[slot, j], grad_hbm.at[pl.ds(row0, R), j], sem_g).wait()


def _body(xr_ref, xrn_ref, xf_ref, tcol_ref, trow_ref, loss_hbm, grad_hbm,
          ls_ref, gs_ref, sim_sc, sem_l, sem_g):
    i = pl.program_id(0)
    nsteps = pl.num_programs(0)
    slot = jax.lax.rem(i, 2)

    @pl.when(i >= 2)
    def _():
        _wait_step(i - 2, loss_hbm, grad_hbm, ls_ref, gs_ref, sem_l, sem_g)

    xf = xf_ref[...]

    @pl.when(i == 0)
    def _():
        sim_sc[0] = jax.lax.dot_general(
            xr_ref[...], xf, (((1,), (1,)), ((), ())),
            preferred_element_type=jnp.float32)

    sim = sim_sc[slot]

    @pl.when(i < nsteps - 1)
    def _():
        sim_sc[jax.lax.rem(i + 1, 2)] = jax.lax.dot_general(
            xrn_ref[...], xf, (((1,), (1,)), ((), ())),
            preferred_element_type=jnp.float32)

    same = tcol_ref[...] == trow_ref[...]

    inf = jnp.float32(jnp.inf)
    min_pos = jnp.min(
        jnp.where(same & (sim < 1.0), sim, inf), axis=1, keepdims=True)
    max_neg = jnp.max(jnp.where(same, -inf, sim), axis=1, keepdims=True)

    thr_p = jnp.minimum(jnp.float32(1.0), max_neg + 0.1)
    thr_n = min_pos - 0.1

    one = jnp.float32(1.0)
    zero = jnp.float32(0.0)
    c1 = jnp.where(sim < thr_p, one, zero)
    c2 = jnp.where(sim > thr_n, one, zero)
    keep_f = jnp.where(same, c1, c2)
    u_pos = jnp.where(same, keep_f, zero)
    ones_mat = jnp.ones((N, 128), jnp.float32)
    sums_all = jax.lax.dot_general(
        keep_f, ones_mat, (((1,), (0,)), ((), ())),
        preferred_element_type=jnp.float32)[:, 0:1]
    sums_pos = jax.lax.dot_general(
        u_pos, ones_mat, (((1,), (0,)), ((), ())),
        preferred_element_type=jnp.float32)[:, 0:1]
    pos_cnt = sums_pos
    neg_cnt = sums_all - sums_pos
    valid = (pos_cnt > zero) & (neg_cnt > zero)

    sp = jnp.where(valid, one, zero)
    sn = jnp.where(valid, jnp.float32(2.0 / ALPHA), zero)
    gp = jnp.where(valid, -2.0 / jnp.maximum(pos_cnt, one), zero)
    gn = jnp.where(valid, 2.0 / jnp.maximum(neg_cnt, one), zero)

    coeff = jnp.where(same, jnp.float32(-BETA), jnp.float32(ALPHA))
    z = coeff * (sim - MARGIN)
    az = jnp.abs(z)
    e = jnp.exp2(az * jnp.float32(-1.4426950408889634))
    lae = jnp.maximum(z, zero) + jnp.log(one + e)
    sig = 0.5 + 0.5 * jnp.tanh(0.5 * z)

    loss_val = keep_f * (jnp.where(same, sp, sn) * lae)
    grad_val = keep_f * (jnp.where(same, gp, gn) * sig)

    for j in range(NJ):
        ls_ref[slot, j] = loss_val[:, 128 * j:128 * (j + 1)]
        gs_ref[slot, j] = grad_val[:, 128 * j:128 * (j + 1)]

    row0 = i * R
    for j in range(NJ):
        pltpu.make_async_copy(
            ls_ref.at[slot, j], loss_hbm.at[pl.ds(row0, R), j], sem_l).start()
        pltpu.make_async_copy(
            gs_ref.at[slot, j], grad_hbm.at[pl.ds(row0, R), j], sem_g).start()

    @pl.when(i == nsteps - 1)
    def _():
        _wait_step(i - 1, loss_hbm, grad_hbm, ls_ref, gs_ref, sem_l, sem_g)
        _wait_step(i, loss_hbm, grad_hbm, ls_ref, gs_ref, sem_l, sem_g)


@jax.jit
def kernel(inputs, targets):
    tcol = targets.reshape(N, 1)
    trow = targets.reshape(1, N)
    grid = (N // R,)
    loss, grad = pl.pallas_call(
        _body,
        grid=grid,
        in_specs=[
            pl.BlockSpec((R, D), lambda i: (i, 0)),
            pl.BlockSpec((R, D), lambda i: (jnp.minimum(i + 1, N // R - 1), 0)),
            pl.BlockSpec((N, D), lambda i: (0, 0)),
            pl.BlockSpec((R, 1), lambda i: (i, 0)),
            pl.BlockSpec((1, N), lambda i: (0, 0)),
        ],
        out_specs=[
            pl.BlockSpec(memory_space=pltpu.MemorySpace.HBM),
            pl.BlockSpec(memory_space=pltpu.MemorySpace.HBM),
        ],
        out_shape=[
            jax.ShapeDtypeStruct((N, NJ, 128), jnp.float32),
            jax.ShapeDtypeStruct((N, NJ, 128), jnp.float32),
        ],
        scratch_shapes=[
            pltpu.VMEM((2, NJ, R, 128), jnp.float32),
            pltpu.VMEM((2, NJ, R, 128), jnp.float32),
            pltpu.VMEM((2, R, N), jnp.float32),
            pltpu.SemaphoreType.DMA,
            pltpu.SemaphoreType.DMA,
        ],
        compiler_params=pltpu.CompilerParams(
            dimension_semantics=("arbitrary",),
        ),
    )(inputs, inputs, inputs, tcol, trow)
    return loss.reshape(-1), grad.reshape(-1)


# speedup vs baseline: 1.1544x; 1.1544x over previous
import jax
import jax.numpy as jnp
from jax.experimental import pallas as pl
from jax.experimental.pallas import tpu as pltpu

N = 4096
D = 128
ALPHA = 40.0
BETA = 2.0
MARGIN = 0.5

R = 256
NJ = N // 128


def _wait_step(step, loss_hbm, grad_hbm, ls_ref, gs_ref, sem_l, sem_g):
    slot = jax.lax.rem(step, 2)
    row0 = step * R
    for j in range(NJ):
        pltpu.make_async_copy(
            ls_ref.at[slot, j], loss_hbm.at[pl.ds(row0, R), j], sem_l).wait()
        pltpu.make_async_copy(
            gs_ref.at[slot, j], grad_hbm.at[pl.ds(row0, R), j], sem_g).wait()


def _body(xr_ref, xf_ref, tcol_ref, trow_ref, loss_hbm, grad_hbm,
          ls_ref, gs_ref, sem_l, sem_g):
    i = pl.program_id(0)
    nsteps = pl.num_programs(0)
    slot = jax.lax.rem(i, 2)

    @pl.when(i >= 2)
    def _():
        _wait_step(i - 2, loss_hbm, grad_hbm, ls_ref, gs_ref, sem_l, sem_g)

    xr = xr_ref[...]
    xf = xf_ref[...]
    sim = jax.lax.dot_general(
        xr, xf, (((1,), (1,)), ((), ())),
        preferred_element_type=jnp.float32)

    same = tcol_ref[...] == trow_ref[...]

    inf = jnp.float32(jnp.inf)
    min_pos = jnp.min(
        jnp.where(same & (sim < 1.0), sim, inf), axis=1, keepdims=True)
    max_neg = jnp.max(jnp.where(same, -inf, sim), axis=1, keepdims=True)

    thr_p = jnp.minimum(jnp.float32(1.0), max_neg + 0.1)
    thr_n = min_pos - 0.1

    one = jnp.float32(1.0)
    zero = jnp.float32(0.0)
    c1 = jnp.where(sim < thr_p, one, zero)
    c2 = jnp.where(sim > thr_n, one, zero)
    keep_f = jnp.where(same, c1, c2)
    u_pos = jnp.where(same, keep_f, zero)
    ones_mat = jnp.ones((N, 128), jnp.float32)
    sums_all = jax.lax.dot_general(
        keep_f, ones_mat, (((1,), (0,)), ((), ())),
        preferred_element_type=jnp.float32)[:, 0:1]
    sums_pos = jax.lax.dot_general(
        u_pos, ones_mat, (((1,), (0,)), ((), ())),
        preferred_element_type=jnp.float32)[:, 0:1]
    pos_cnt = sums_pos
    neg_cnt = sums_all - sums_pos
    valid = (pos_cnt > zero) & (neg_cnt > zero)

    sp = jnp.where(valid, one, zero)
    sn = jnp.where(valid, jnp.float32(2.0 / ALPHA), zero)
    gp = jnp.where(valid, -2.0 / jnp.maximum(pos_cnt, one), zero)
    gn = jnp.where(valid, 2.0 / jnp.maximum(neg_cnt, one), zero)

    coeff = jnp.where(same, jnp.float32(-BETA), jnp.float32(ALPHA))
    z = coeff * (sim - MARGIN)
    az = jnp.abs(z)
    e = jnp.exp2(az * jnp.float32(-1.4426950408889634))
    lae = jnp.maximum(z, zero) + jnp.log(one + e)
    sig = 0.5 + 0.5 * jnp.tanh(0.5 * z)

    loss_val = keep_f * (jnp.where(same, sp, sn) * lae)
    grad_val = keep_f * (jnp.where(same, gp, gn) * sig)

    for j in range(NJ):
        ls_ref[slot, j] = loss_val[:, 128 * j:128 * (j + 1)]
        gs_ref[slot, j] = grad_val[:, 128 * j:128 * (j + 1)]

    row0 = i * R
    for j in range(NJ):
        pltpu.make_async_copy(
            ls_ref.at[slot, j], loss_hbm.at[pl.ds(row0, R), j], sem_l).start()
        pltpu.make_async_copy(
            gs_ref.at[slot, j], grad_hbm.at[pl.ds(row0, R), j], sem_g).start()

    @pl.when(i == nsteps - 1)
    def _():
        _wait_step(i - 1, loss_hbm, grad_hbm, ls_ref, gs_ref, sem_l, sem_g)
        _wait_step(i, loss_hbm, grad_hbm, ls_ref, gs_ref, sem_l, sem_g)


@jax.jit
def kernel(inputs, targets):
    tcol = targets.reshape(N, 1)
    trow = targets.reshape(1, N)
    grid = (N // R,)
    loss, grad = pl.pallas_call(
        _body,
        grid=grid,
        in_specs=[
            pl.BlockSpec((R, D), lambda i: (i, 0)),
            pl.BlockSpec((N, D), lambda i: (0, 0)),
            pl.BlockSpec((R, 1), lambda i: (i, 0)),
            pl.BlockSpec((1, N), lambda i: (0, 0)),
        ],
        out_specs=[
            pl.BlockSpec(memory_space=pltpu.MemorySpace.HBM),
            pl.BlockSpec(memory_space=pltpu.MemorySpace.HBM),
        ],
        out_shape=[
            jax.ShapeDtypeStruct((N, NJ, 128), jnp.float32),
            jax.ShapeDtypeStruct((N, NJ, 128), jnp.float32),
        ],
        scratch_shapes=[
            pltpu.VMEM((2, NJ, R, 128), jnp.float32),
            pltpu.VMEM((2, NJ, R, 128), jnp.float32),
            pltpu.SemaphoreType.DMA,
            pltpu.SemaphoreType.DMA,
        ],
        compiler_params=pltpu.CompilerParams(
            dimension_semantics=("arbitrary",),
        ),
    )(inputs, inputs, tcol, trow)
    return loss.reshape(-1), grad.reshape(-1)
